# baseline (speedup 1.0000x reference)
import functools
import math

import jax
import jax.numpy as jnp
from jax import lax
from jax.experimental import pallas as pl
from jax.experimental.pallas import tpu as pltpu
from jax.experimental.pallas import tpu_sc as plsc

_BATCH = 16384
_LANES = 16

_MESH = plsc.VectorSubcoreMesh(
    core_axis_name="c", subcore_axis_name="s", num_cores=1)
_NW = _MESH.num_subcores
_BPW = _BATCH // _NW
_ROWS = _BPW // 128
_LN2 = float(math.log(2.0))


def _vlog(w):
    bits = lax.bitcast_convert_type(w, jnp.int32)
    k = (bits >> 23) - 127
    m = lax.bitcast_convert_type((bits & 0x7FFFFF) | 0x3F800000, jnp.float32)
    t = (m - 1.0) / (m + 1.0)
    t2 = t * t
    poly = 1.0 + t2 * (1.0 / 3.0 + t2 * (1.0 / 5.0 + t2 * (1.0 / 7.0 + t2 * (1.0 / 9.0))))
    return k.astype(jnp.float32) * _LN2 + 2.0 * t * poly


@functools.partial(
    pl.kernel,
    out_type=jax.ShapeDtypeStruct((_LANES,), jnp.float32),
    mesh=_MESH,
    scratch_types=[
        pltpu.VMEM((_ROWS, 128), jnp.int32),
        pltpu.VMEM((_ROWS, 128), jnp.int32),
        pltpu.VMEM((_ROWS, 128), jnp.float32),
        pltpu.VMEM((_ROWS, 128), jnp.float32),
        pltpu.VMEM((_ROWS, 128), jnp.float32),
        pltpu.VMEM((_ROWS, 128), jnp.float32),
        pltpu.VMEM((_LANES,), jnp.float32),
        pltpu.VMEM((_NW * _LANES,), jnp.float32),
        pltpu.VMEM_SHARED((_NW * _LANES,), jnp.float32),
        pltpu.SemaphoreType.DMA,
    ],
)
def _sc_irt_loss(u_hbm, i_hbm, s_hbm, theta_hbm, diff_hbm, disc_hbm, out_hbm,
                 u_v, i_v, s_v, th_v, df_v, dc_v, acc_v, red_v, shared_v, sem):
    wid = lax.axis_index("s")
    pltpu.sync_copy(u_hbm.at[wid], u_v)
    pltpu.sync_copy(i_hbm.at[wid], i_v)
    pltpu.sync_copy(s_hbm.at[wid], s_v)
    copies = []
    for j in range(_ROWS):
        copies.append(pltpu.async_copy(theta_hbm.at[u_v.at[j]], th_v.at[j], sem))
        copies.append(pltpu.async_copy(diff_hbm.at[i_v.at[j]], df_v.at[j], sem))
        copies.append(pltpu.async_copy(disc_hbm.at[i_v.at[j]], dc_v.at[j], sem))
    for c in copies:
        c.wait()

    acc = jnp.zeros((_LANES,), jnp.float32)
    for j in range(_ROWS):
        for k in range(128 // _LANES):
            sl = pl.ds(k * _LANES, _LANES)
            z = 1.702 * dc_v[j, sl] * (th_v[j, sl] - df_v[j, sl])
            w = 1.0 + jnp.exp(-z)
            acc = acc + _vlog(w) + (1.0 - s_v[j, sl]) * z
    acc_v[...] = acc
    pltpu.sync_copy(acc_v, shared_v.at[pl.ds(wid * _LANES, _LANES)])
    plsc.subcore_barrier()

    @pl.when(wid == 0)
    def _():
        pltpu.sync_copy(shared_v, red_v)
        tot = red_v[pl.ds(0, _LANES)]
        for r in range(1, _NW):
            tot = tot + red_v[pl.ds(r * _LANES, _LANES)]
        for sh in (8, 4, 2, 1):
            idx = (lax.iota(jnp.int32, _LANES) + sh) & (_LANES - 1)
            tot = tot + tot.at[idx].get(mode="promise_in_bounds")
        acc_v[...] = tot * (1.0 / _BATCH)
        pltpu.sync_copy(acc_v, out_hbm)


def kernel(u, i, s, diff, disc, theta):
    u3 = u.astype(jnp.int32).reshape(_NW, _ROWS, 128)
    i3 = i.astype(jnp.int32).reshape(_NW, _ROWS, 128)
    s3 = s.astype(jnp.float32).reshape(_NW, _ROWS, 128)
    out = _sc_irt_loss(u3, i3, s3,
                       theta.reshape(-1).astype(jnp.float32),
                       diff.reshape(-1).astype(jnp.float32),
                       disc.reshape(-1).astype(jnp.float32))
    return out[0]


# speedup vs baseline: 1.0690x; 1.0690x over previous
import functools

import jax
import jax.numpy as jnp
from jax import lax
from jax.experimental import pallas as pl
from jax.experimental.pallas import tpu as pltpu
from jax.experimental.pallas import tpu_sc as plsc

_BATCH = 16384
_LANES = 16

_MESH = plsc.VectorSubcoreMesh(core_axis_name="c", subcore_axis_name="s")
_NC = _MESH.num_cores
_NS = _MESH.num_subcores
_NW = _NC * _NS
_BPW = _BATCH // _NW
_ROWS = _BPW // 128


@functools.partial(
    pl.kernel,
    out_type=jax.ShapeDtypeStruct((_NW, _ROWS, 128), jnp.float32),
    mesh=_MESH,
    scratch_types=[
        pltpu.VMEM((_ROWS, 128), jnp.int32),
        pltpu.VMEM((_ROWS, 128), jnp.int32),
        pltpu.VMEM((_ROWS, 128), jnp.float32),
        pltpu.VMEM((_ROWS, 128), jnp.float32),
        pltpu.VMEM((_ROWS, 128), jnp.float32),
        pltpu.VMEM((_ROWS, 128), jnp.float32),
        pltpu.SemaphoreType.DMA,
    ],
)
def _sc_gather_logit(u_hbm, i_hbm, theta_hbm, diff_hbm, disc_hbm, z_hbm,
                     u_v, i_v, th_v, df_v, dc_v, z_v, sem):
    wid = lax.axis_index("s") * _NC + lax.axis_index("c")
    pltpu.sync_copy(u_hbm.at[wid], u_v)
    pltpu.sync_copy(i_hbm.at[wid], i_v)
    copies = []
    for j in range(_ROWS):
        copies.append(pltpu.async_copy(theta_hbm.at[u_v.at[j]], th_v.at[j], sem))
        copies.append(pltpu.async_copy(diff_hbm.at[i_v.at[j]], df_v.at[j], sem))
        copies.append(pltpu.async_copy(disc_hbm.at[i_v.at[j]], dc_v.at[j], sem))
    for c in copies:
        c.wait()
    for j in range(_ROWS):
        for k in range(128 // _LANES):
            sl = pl.ds(k * _LANES, _LANES)
            z_v[j, sl] = 1.702 * dc_v[j, sl] * (th_v[j, sl] - df_v[j, sl])
    pltpu.sync_copy(z_v, z_hbm.at[wid])


def _loss_body(z_ref, s_ref, o_ref):
    z = z_ref[...]
    pred = 1.0 / (1.0 + jnp.exp(-z))
    p = jnp.clip(pred, 1e-12, 1.0 - 1e-12)
    s = s_ref[...]
    bce = s * jnp.log(p) + (1.0 - s) * jnp.log(1.0 - p)
    o_ref[...] = jnp.reshape(-jnp.sum(bce) * (1.0 / _BATCH), (1, 1))


_tc_loss = pl.pallas_call(
    _loss_body,
    out_shape=jax.ShapeDtypeStruct((1, 1), jnp.float32),
)


def kernel(u, i, s, diff, disc, theta):
    u3 = u.astype(jnp.int32).reshape(_NW, _ROWS, 128)
    i3 = i.astype(jnp.int32).reshape(_NW, _ROWS, 128)
    z = _sc_gather_logit(u3, i3,
                         theta.reshape(-1).astype(jnp.float32),
                         diff.reshape(-1).astype(jnp.float32),
                         disc.reshape(-1).astype(jnp.float32))
    z2 = z.reshape(128, 128)
    s2 = s.astype(jnp.float32).reshape(128, 128)
    return _tc_loss(z2, s2)[0, 0]
